# baseline (speedup 1.0000x reference)
import functools

import jax
import jax.numpy as jnp
from jax.experimental import pallas as pl
from jax.experimental.pallas import tpu as pltpu

B = 1
S = 2048
HIDDEN = 2048
NH = 16
NKV = 8
HD = 128
EPS = 1e-6
SCALE = HD ** -0.5

BQ = 256
NEG = -1e30


def _rope(x, cos, sin):
    x1 = x[:, : HD // 2]
    x2 = x[:, HD // 2:]
    rot = jnp.concatenate([-x2, x1], axis=1)
    return x * cos + rot * sin


def _rms_norm_head(x, w):
    var = jnp.mean(x * x, axis=-1, keepdims=True)
    return x * jax.lax.rsqrt(var + EPS) * w


def _qkv_kernel(hs_ref, wq_ref, wk_ref, wv_ref, cos_ref, sin_ref,
                qw_ref, kw_ref, q_out, k_out, v_out):
    x = hs_ref[...]
    cos = cos_ref[...]
    sin = sin_ref[...]
    qw = qw_ref[...]
    kw = kw_ref[...]

    q = jnp.dot(x, wq_ref[...], preferred_element_type=jnp.float32)
    for h in range(NH):
        qh = q[:, h * HD:(h + 1) * HD]
        qh = _rms_norm_head(qh, qw)
        q_out[h] = _rope(qh, cos, sin)

    k = jnp.dot(x, wk_ref[...], preferred_element_type=jnp.float32)
    for h in range(NKV):
        kh = k[:, h * HD:(h + 1) * HD]
        kh = _rms_norm_head(kh, kw)
        k_out[h] = _rope(kh, cos, sin)

    v = jnp.dot(x, wv_ref[...], preferred_element_type=jnp.float32)
    for h in range(NKV):
        v_out[h] = v[:, h * HD:(h + 1) * HD]


def _attn_kernel(q_ref, k_ref, v_ref, wo_ref, out_ref,
                 acc_ref, m_ref, l_ref):
    i = pl.program_id(0)
    h = pl.program_id(1)
    hkv = h // (NH // NKV)

    qh = q_ref[0]

    m_ref[...] = jnp.full((BQ, 1), NEG, jnp.float32)
    l_ref[...] = jnp.zeros((BQ, 1), jnp.float32)
    acc_ref[...] = jnp.zeros((BQ, HD), jnp.float32)

    row = jax.lax.broadcasted_iota(jnp.int32, (BQ, BQ), 0)
    col = jax.lax.broadcasted_iota(jnp.int32, (BQ, BQ), 1)

    def body(j, _):
        kj = k_ref[hkv, pl.ds(j * BQ, BQ), :]
        vj = v_ref[hkv, pl.ds(j * BQ, BQ), :]
        s = jax.lax.dot_general(
            qh, kj, (((1,), (1,)), ((), ())),
            preferred_element_type=jnp.float32) * SCALE
        mask = (col + j * BQ) <= (row + i * BQ)
        s = jnp.where(mask, s, NEG)

        m_old = m_ref[...]
        m_new = jnp.maximum(m_old, jnp.max(s, axis=1, keepdims=True))
        alpha = jnp.exp(m_old - m_new)
        p = jnp.exp(s - m_new)
        l_ref[...] = l_ref[...] * alpha + jnp.sum(p, axis=1, keepdims=True)
        pv = jnp.dot(p, vj, preferred_element_type=jnp.float32)
        acc_ref[...] = acc_ref[...] * alpha + pv
        m_ref[...] = m_new
        return 0

    jax.lax.fori_loop(0, i + 1, body, 0)

    out_h = acc_ref[...] / l_ref[...]
    wo_h = wo_ref[pl.ds(h * HD, HD), :]
    partial = jnp.dot(out_h, wo_h, preferred_element_type=jnp.float32)

    @pl.when(h == 0)
    def _():
        out_ref[...] = partial

    @pl.when(h > 0)
    def _():
        out_ref[...] += partial


@jax.jit
def kernel(hidden_states, cos, sin, Wq, Wk, Wv, Wo, q_norm_w, k_norm_w):
    hs = hidden_states.reshape(S, HIDDEN)
    cos2 = cos.reshape(S, HD)
    sin2 = sin.reshape(S, HD)
    qw = q_norm_w.reshape(1, HD)
    kw = k_norm_w.reshape(1, HD)

    nblk = S // BQ

    q, k, v = pl.pallas_call(
        _qkv_kernel,
        grid=(nblk,),
        in_specs=[
            pl.BlockSpec((BQ, HIDDEN), lambda i: (i, 0)),
            pl.BlockSpec((HIDDEN, NH * HD), lambda i: (0, 0)),
            pl.BlockSpec((HIDDEN, NKV * HD), lambda i: (0, 0)),
            pl.BlockSpec((HIDDEN, NKV * HD), lambda i: (0, 0)),
            pl.BlockSpec((BQ, HD), lambda i: (i, 0)),
            pl.BlockSpec((BQ, HD), lambda i: (i, 0)),
            pl.BlockSpec((1, HD), lambda i: (0, 0)),
            pl.BlockSpec((1, HD), lambda i: (0, 0)),
        ],
        out_specs=[
            pl.BlockSpec((NH, BQ, HD), lambda i: (0, i, 0)),
            pl.BlockSpec((NKV, BQ, HD), lambda i: (0, i, 0)),
            pl.BlockSpec((NKV, BQ, HD), lambda i: (0, i, 0)),
        ],
        out_shape=[
            jax.ShapeDtypeStruct((NH, S, HD), jnp.float32),
            jax.ShapeDtypeStruct((NKV, S, HD), jnp.float32),
            jax.ShapeDtypeStruct((NKV, S, HD), jnp.float32),
        ],
    )(hs, Wq, Wk, Wv, cos2, sin2, qw, kw)

    out = pl.pallas_call(
        _attn_kernel,
        grid=(nblk, NH),
        in_specs=[
            pl.BlockSpec((1, BQ, HD), lambda i, h: (h, i, 0)),
            pl.BlockSpec((NKV, S, HD), lambda i, h: (0, 0, 0)),
            pl.BlockSpec((NKV, S, HD), lambda i, h: (0, 0, 0)),
            pl.BlockSpec((NH * HD, HIDDEN), lambda i, h: (0, 0)),
        ],
        out_specs=pl.BlockSpec((BQ, HIDDEN), lambda i, h: (i, 0)),
        out_shape=jax.ShapeDtypeStruct((S, HIDDEN), jnp.float32),
        scratch_shapes=[
            pltpu.VMEM((BQ, HD), jnp.float32),
            pltpu.VMEM((BQ, 1), jnp.float32),
            pltpu.VMEM((BQ, 1), jnp.float32),
        ],
    )(q, k, v, Wo)

    return out.reshape(B, S, HIDDEN)
